# baseline (speedup 1.0000x reference)
import functools

import jax
import jax.numpy as jnp
from jax import lax
from jax.experimental import pallas as pl
from jax.experimental.pallas import tpu as pltpu
from jax.experimental.pallas import tpu_sc as plsc

N = 10000
D_IN = 128
HID = 128
NCLS = 64
E = 320000
TE = 320000

NC = 2
NS = 16
NW = NC * NS
CHUNK = 128
ECH = 80
KBUF = 1
E_PAD = NW * ECH * CHUNK
N_PAD = 10112
ROWS_PER_TILE = N_PAD // NS
E_PER_TILE = E_PAD // NW
TE_PAD = E_PAD
TE_PER_TILE = TE_PAD // NW

_MESH = plsc.VectorSubcoreMesh(
    core_axis_name="c", subcore_axis_name="s", num_cores=NC, num_subcores=NS)

_SC_PARAMS = pltpu.CompilerParams(needs_layout_passes=False,
                                  use_tc_tiling_on_sc=False)


@functools.partial(
    pl.kernel,
    out_type=jax.ShapeDtypeStruct((NW * N_PAD,), jnp.float32),
    mesh=_MESH,
    compiler_params=_SC_PARAMS,
    scratch_types=[
        pltpu.VMEM((E_PER_TILE,), jnp.int32),
        pltpu.VMEM((N_PAD,), jnp.float32),
    ],
)
def _deg_kernel(dst_hbm, out_hbm, idx_v, cnt_v):
    c = lax.axis_index("c")
    s = lax.axis_index("s")
    w = c * NS + s
    pltpu.sync_copy(dst_hbm.at[pl.ds(w * E_PER_TILE, E_PER_TILE)], idx_v)
    zero16 = jnp.zeros((16,), jnp.float32)

    def zbody(i, _):
        cnt_v[pl.ds(i * 16, 16)] = zero16
        return 0

    lax.fori_loop(0, N_PAD // 16, zbody, 0)
    one16 = jnp.ones((16,), jnp.float32)

    def body(i, _):
        idx = idx_v[pl.ds(i * 16, 16)]
        plsc.addupdate_scatter(cnt_v, [idx], one16)
        return 0

    lax.fori_loop(0, E_PER_TILE // 16, body, 0)
    pltpu.sync_copy(cnt_v, out_hbm.at[pl.ds(w * N_PAD, N_PAD)])


def _make_gather_scatter(D, nbuf):
    scratch = ([pltpu.VMEM((ECH, CHUNK), jnp.int32),
                pltpu.VMEM((ECH, CHUNK), jnp.int32)]
               + [pltpu.VMEM((CHUNK, D), jnp.float32) for _ in range(nbuf)]
               + [pltpu.VMEM_SHARED((N_PAD, D), jnp.float32)]
               + [pltpu.SemaphoreType.DMA for _ in range(2 * nbuf)])

    @functools.partial(
        pl.kernel,
        out_type=jax.ShapeDtypeStruct((NC, N_PAD, D), jnp.float32),
        mesh=_MESH,
        compiler_params=_SC_PARAMS,
        scratch_types=scratch,
    )
    def k(src_hbm, dst_hbm, h_hbm, zeros_hbm, out_hbm, src_v, dst_v, *rest):
        rows = rest[:nbuf]
        acc = rest[nbuf]
        sems = rest[nbuf + 1:]
        c = lax.axis_index("c")
        s = lax.axis_index("s")
        pltpu.sync_copy(src_hbm.at[c, s], src_v)
        pltpu.sync_copy(dst_hbm.at[c, s], dst_v)
        row0 = s * ROWS_PER_TILE
        pltpu.sync_copy(zeros_hbm.at[pl.ds(row0, ROWS_PER_TILE)],
                        acc.at[pl.ds(row0, ROWS_PER_TILE)])
        plsc.subcore_barrier()

        def body(j, _):
            pltpu.async_copy(h_hbm.at[src_v.at[j]], rows[0], sems[0]).wait()
            pltpu.sync_copy(rows[0], acc.at[dst_v.at[j]], add=True)
            return 0

        lax.fori_loop(0, ECH, body, 0)
        plsc.subcore_barrier()
        pltpu.sync_copy(acc.at[pl.ds(row0, ROWS_PER_TILE)],
                        out_hbm.at[c, pl.ds(row0, ROWS_PER_TILE)])

    return k


_gs_hid = _make_gather_scatter(HID, 1)
_gs_cls = _make_gather_scatter(NCLS, 1)


def _make_dist(D):
    scratch = ([pltpu.VMEM((ECH, CHUNK), jnp.int32),
                pltpu.VMEM((ECH, CHUNK), jnp.int32)]
               + [pltpu.VMEM((CHUNK, D), jnp.float32) for _ in range(KBUF)]
               + [pltpu.SemaphoreType.DMA for _ in range(3 * KBUF)])

    @functools.partial(
        pl.kernel,
        out_type=jax.ShapeDtypeStruct((TE_PAD, D), jnp.float32),
        mesh=_MESH,
        compiler_params=_SC_PARAMS,
        scratch_types=scratch,
    )
    def k(pos_hbm, neg_hbm, t0_hbm, t1_hbm, out_hbm, t0_v, t1_v, *rest):
        rows = rest[:KBUF]
        semg = rest[KBUF:2 * KBUF]
        sema = rest[2 * KBUF:3 * KBUF]
        semw = rest[3 * KBUF:4 * KBUF]
        c = lax.axis_index("c")
        s = lax.axis_index("s")
        w = c * NS + s
        base = w * E_PER_TILE

        pltpu.sync_copy(t0_hbm.at[c, s], t0_v)
        pltpu.sync_copy(t1_hbm.at[c, s], t1_v)

        def body(j, _):
            pltpu.async_copy(pos_hbm.at[t0_v.at[j]], rows[0], semg[0]).wait()
            pltpu.async_copy(neg_hbm.at[t1_v.at[j]], rows[0], sema[0],
                             add=True).wait()
            pltpu.sync_copy(rows[0],
                            out_hbm.at[pl.ds(base + j * CHUNK, CHUNK)])
            return 0

        lax.fori_loop(0, ECH, body, 0)

    return k


_dist_hid = _make_dist(HID)
_dist_cls = _make_dist(NCLS)


def _tc1_body(x_ref, w1_ref, cnt_ref, hp_ref, xneg_ref, dinv_ref):
    cs = jnp.sum(cnt_ref[...], axis=0)
    deg = cs[:N] + 1.0
    y = lax.rsqrt(deg)
    dinv = y * (1.5 - 0.5 * deg * y * y)
    dinv_ref[...] = dinv
    xx = x_ref[...]
    xneg_ref[...] = -xx
    h = jnp.dot(xx.astype(jnp.bfloat16), w1_ref[...].astype(jnp.bfloat16),
                preferred_element_type=jnp.float32)
    hp_ref[...] = h * dinv[:, None]


def _tc3_body(p_ref, hp_ref, dinv_ref, b1_ref, w2_ref,
              h1_ref, h1neg_ref, h2p_ref):
    agg = p_ref[0, :N, :] + p_ref[1, :N, :] + hp_ref[...]
    dinv = dinv_ref[...]
    h1 = jnp.maximum(agg * dinv[:, None] + b1_ref[...][None, :], 0.0)
    h1_ref[...] = h1
    h1neg_ref[...] = -h1
    h2r = jnp.dot(h1.astype(jnp.bfloat16), w2_ref[...].astype(jnp.bfloat16),
                  preferred_element_type=jnp.float32)
    h2p_ref[...] = h2r * dinv[:, None]


def _tc4_body(p_ref, h2p_ref, dinv_ref, b2_ref, h2_ref, h2neg_ref):
    agg = p_ref[0, :N, :] + p_ref[1, :N, :] + h2p_ref[...]
    h2 = agg * dinv_ref[...][:, None] + b2_ref[...][None, :]
    h2_ref[...] = h2
    h2neg_ref[...] = -h2


_tc1 = pl.pallas_call(
    _tc1_body,
    out_shape=(jax.ShapeDtypeStruct((N, HID), jnp.float32),
               jax.ShapeDtypeStruct((N, D_IN), jnp.float32),
               jax.ShapeDtypeStruct((N,), jnp.float32)),
)

_tc3 = pl.pallas_call(
    _tc3_body,
    out_shape=(jax.ShapeDtypeStruct((N, HID), jnp.float32),
               jax.ShapeDtypeStruct((N, HID), jnp.float32),
               jax.ShapeDtypeStruct((N, NCLS), jnp.float32)),
)

_tc4 = pl.pallas_call(
    _tc4_body,
    out_shape=(jax.ShapeDtypeStruct((N, NCLS), jnp.float32),
               jax.ShapeDtypeStruct((N, NCLS), jnp.float32)),
)

EB = 4096
NB = TE_PAD // EB


def _tc5_body(d1_ref, d2_ref, d3_ref, s1w_ref, s2w_ref, s3w_ref, s3b_ref,
              o3_ref, sum_ref):
    b = pl.program_id(0)
    d1 = d1_ref[...].astype(jnp.bfloat16)
    d2 = d2_ref[...].astype(jnp.bfloat16)
    d3 = d3_ref[...].astype(jnp.bfloat16)
    w1 = s1w_ref[...].astype(jnp.bfloat16)[:, None]
    w2 = s2w_ref[...].astype(jnp.bfloat16)[:, None]
    w3 = s3w_ref[...].astype(jnp.bfloat16)[:, None]
    e1 = jnp.dot(d1, w1, preferred_element_type=jnp.float32)[:, 0]
    e2 = jnp.dot(d2, w2, preferred_element_type=jnp.float32)[:, 0]
    e3 = jnp.dot(d3, w3, preferred_element_type=jnp.float32)[:, 0]
    o3_ref[...] = jnp.maximum(e3 + s3b_ref[...], 0.0)
    gid = b * EB + lax.broadcasted_iota(jnp.int32, (EB,), 0)
    part = jnp.sum(jnp.where(gid < TE, e1 + e2 + e3, 0.0))

    @pl.when(b == 0)
    def _():
        sum_ref[...] = jnp.zeros_like(sum_ref)

    sum_ref[...] += jnp.full((8, 128), part, jnp.float32)


_tc5 = pl.pallas_call(
    _tc5_body,
    grid=(NB,),
    in_specs=[
        pl.BlockSpec((EB, HID), lambda b: (b, 0)),
        pl.BlockSpec((EB, HID), lambda b: (b, 0)),
        pl.BlockSpec((EB, NCLS), lambda b: (b, 0)),
        pl.BlockSpec((D_IN,), lambda b: (0,)),
        pl.BlockSpec((HID,), lambda b: (0,)),
        pl.BlockSpec((NCLS,), lambda b: (0,)),
        pl.BlockSpec((1,), lambda b: (0,)),
    ],
    out_specs=(pl.BlockSpec((EB,), lambda b: (b,)),
               pl.BlockSpec((8, 128), lambda b: (0, 0))),
    out_shape=(jax.ShapeDtypeStruct((TE_PAD,), jnp.float32),
               jax.ShapeDtypeStruct((8, 128), jnp.float32)),
)


def kernel(x, masked_nodes, pos_edge_index, neg_edge_index, edge_index,
           W1, b1, W2, b2, s1w, s1b, s2w, s2b, s3w, s3b):
    src = edge_index[0].astype(jnp.int32)
    dst = edge_index[1].astype(jnp.int32)
    pad_src = jnp.zeros((E_PAD - E,), jnp.int32)
    pad_dst = jnp.full((E_PAD - E,), N, jnp.int32)
    src_p = jnp.concatenate([src, pad_src]).reshape(NC, NS, ECH, CHUNK)
    dst_flat = jnp.concatenate([dst, pad_dst])
    dst_p = dst_flat.reshape(NC, NS, ECH, CHUNK)
    pad_t = jnp.zeros((TE_PAD - TE,), jnp.int32)
    t0 = jnp.concatenate(
        [pos_edge_index[0], neg_edge_index[0], pad_t]).astype(jnp.int32)
    t1 = jnp.concatenate(
        [pos_edge_index[1], neg_edge_index[1], pad_t]).astype(jnp.int32)
    t0_p = t0.reshape(NC, NS, ECH, CHUNK)
    t1_p = t1.reshape(NC, NS, ECH, CHUNK)

    cnt = _deg_kernel(dst_flat).reshape(NW, N_PAD)
    hp, xneg, dinv = _tc1(x, W1, cnt)
    dist1 = _dist_hid(x, xneg, t0_p, t1_p)
    parts1 = _gs_hid(src_p, dst_p, hp, jnp.zeros((N_PAD, HID), jnp.float32))
    h1, h1neg, h2p = _tc3(parts1, hp, dinv, b1, W2)
    dist2 = _dist_hid(h1, h1neg, t0_p, t1_p)
    parts2 = _gs_cls(src_p, dst_p, h2p, jnp.zeros((N_PAD, NCLS), jnp.float32))
    h2, h2neg = _tc4(parts2, h2p, dinv, b2)
    dist3 = _dist_cls(h2, h2neg, t0_p, t1_p)
    o3_pad, sums = _tc5(dist1, dist2, dist3, s1w, s2w, s3w, s3b)
    score_loss = sums[0, 0] / TE
    return (o3_pad[:TE], score_loss)


# speedup vs baseline: 2.7459x; 2.7459x over previous
import functools

import jax
import jax.numpy as jnp
from jax import lax
from jax.experimental import pallas as pl
from jax.experimental.pallas import tpu as pltpu
from jax.experimental.pallas import tpu_sc as plsc

N = 10000
D_IN = 128
HID = 128
NCLS = 64
E = 320000
TE = 320000

NC = 2
NS = 16
NW = NC * NS
CHUNK = 128
ECH = 80
KBUF = 1
E_PAD = NW * ECH * CHUNK
N_PAD = 10112
ROWS_PER_TILE = N_PAD // NS
E_PER_TILE = E_PAD // NW
TE_PAD = E_PAD
TE_PER_TILE = TE_PAD // NW

_MESH = plsc.VectorSubcoreMesh(
    core_axis_name="c", subcore_axis_name="s", num_cores=NC, num_subcores=NS)

_SC_PARAMS = pltpu.CompilerParams(needs_layout_passes=False,
                                  use_tc_tiling_on_sc=False)


@functools.partial(
    pl.kernel,
    out_type=jax.ShapeDtypeStruct((NW * N_PAD,), jnp.float32),
    mesh=_MESH,
    compiler_params=_SC_PARAMS,
    scratch_types=[
        pltpu.VMEM((E_PER_TILE,), jnp.int32),
        pltpu.VMEM((N_PAD,), jnp.float32),
    ],
)
def _deg_kernel(dst_hbm, out_hbm, idx_v, cnt_v):
    c = lax.axis_index("c")
    s = lax.axis_index("s")
    w = c * NS + s
    pltpu.sync_copy(dst_hbm.at[pl.ds(w * E_PER_TILE, E_PER_TILE)], idx_v)
    zero16 = jnp.zeros((16,), jnp.float32)

    def zbody(i, _):
        cnt_v[pl.ds(i * 16, 16)] = zero16
        return 0

    lax.fori_loop(0, N_PAD // 16, zbody, 0)
    one16 = jnp.ones((16,), jnp.float32)

    def body(i, _):
        idx = idx_v[pl.ds(i * 16, 16)]
        plsc.addupdate_scatter(cnt_v, [idx], one16)
        return 0

    lax.fori_loop(0, E_PER_TILE // 16, body, 0)
    pltpu.sync_copy(cnt_v, out_hbm.at[pl.ds(w * N_PAD, N_PAD)])


def _make_gather_scatter(D, nbuf):
    scratch = ([pltpu.VMEM((ECH, CHUNK), jnp.int32),
                pltpu.VMEM((ECH, CHUNK), jnp.int32)]
               + [pltpu.VMEM((CHUNK, D), jnp.float32) for _ in range(nbuf)]
               + [pltpu.VMEM_SHARED((N_PAD, D), jnp.float32)]
               + [pltpu.SemaphoreType.DMA for _ in range(2 * nbuf)])

    @functools.partial(
        pl.kernel,
        out_type=jax.ShapeDtypeStruct((NC, N_PAD, D), jnp.float32),
        mesh=_MESH,
        compiler_params=_SC_PARAMS,
        scratch_types=scratch,
    )
    def k(src_hbm, dst_hbm, h_hbm, zeros_hbm, out_hbm, src_v, dst_v, *rest):
        rows = rest[:nbuf]
        acc = rest[nbuf]
        sems = rest[nbuf + 1:]
        c = lax.axis_index("c")
        s = lax.axis_index("s")
        pltpu.sync_copy(src_hbm.at[c, s], src_v)
        pltpu.sync_copy(dst_hbm.at[c, s], dst_v)
        row0 = s * ROWS_PER_TILE
        pltpu.sync_copy(zeros_hbm.at[pl.ds(row0, ROWS_PER_TILE)],
                        acc.at[pl.ds(row0, ROWS_PER_TILE)])
        plsc.subcore_barrier()

        def body(j, _):
            pltpu.async_copy(h_hbm.at[src_v.at[j]], rows[0], sems[0]).wait()
            pltpu.sync_copy(rows[0], acc.at[dst_v.at[j]], add=True)
            return 0

        lax.fori_loop(0, ECH, body, 0)
        plsc.subcore_barrier()
        pltpu.sync_copy(acc.at[pl.ds(row0, ROWS_PER_TILE)],
                        out_hbm.at[c, pl.ds(row0, ROWS_PER_TILE)])

    return k


_gs_hid = _make_gather_scatter(HID, 1)
_gs_cls = _make_gather_scatter(NCLS, 1)


def _make_dist(D):
    scratch = ([pltpu.VMEM((ECH, CHUNK), jnp.int32),
                pltpu.VMEM((ECH, CHUNK), jnp.int32)]
               + [pltpu.VMEM((CHUNK, D), jnp.float32) for _ in range(KBUF)]
               + [pltpu.SemaphoreType.DMA for _ in range(3 * KBUF)])

    @functools.partial(
        pl.kernel,
        out_type=jax.ShapeDtypeStruct((TE_PAD, D), jnp.float32),
        mesh=_MESH,
        compiler_params=_SC_PARAMS,
        scratch_types=scratch,
    )
    def k(pos_hbm, neg_hbm, t0_hbm, t1_hbm, out_hbm, t0_v, t1_v, *rest):
        rows = rest[:KBUF]
        semg = rest[KBUF:2 * KBUF]
        sema = rest[2 * KBUF:3 * KBUF]
        semw = rest[3 * KBUF:4 * KBUF]
        c = lax.axis_index("c")
        s = lax.axis_index("s")
        w = c * NS + s
        base = w * E_PER_TILE

        pltpu.sync_copy(t0_hbm.at[c, s], t0_v)
        pltpu.sync_copy(t1_hbm.at[c, s], t1_v)

        def body(j, _):
            pltpu.async_copy(pos_hbm.at[t0_v.at[j]], rows[0], semg[0]).wait()
            pltpu.async_copy(neg_hbm.at[t1_v.at[j]], rows[0], sema[0],
                             add=True).wait()
            pltpu.sync_copy(rows[0],
                            out_hbm.at[pl.ds(base + j * CHUNK, CHUNK)])
            return 0

        lax.fori_loop(0, ECH, body, 0)

    return k


_dist_hid = _make_dist(HID)
_dist_cls = _make_dist(NCLS)


def _tc1_body(x_ref, w1_ref, cnt_ref, hp_ref, xneg_ref, dinv_ref):
    cs = jnp.sum(cnt_ref[...], axis=0)
    deg = cs[:N] + 1.0
    y = lax.rsqrt(deg)
    dinv = y * (1.5 - 0.5 * deg * y * y)
    dinv_ref[...] = dinv
    xx = x_ref[...]
    xneg_ref[...] = -xx
    h = jnp.dot(xx.astype(jnp.bfloat16), w1_ref[...].astype(jnp.bfloat16),
                preferred_element_type=jnp.float32)
    hp_ref[...] = h * dinv[:, None]


def _tc3_body(p_ref, hp_ref, dinv_ref, b1_ref, w2_ref,
              h1_ref, h1neg_ref, h2p_ref):
    agg = p_ref[0, :N, :] + p_ref[1, :N, :] + hp_ref[...]
    dinv = dinv_ref[...]
    h1 = jnp.maximum(agg * dinv[:, None] + b1_ref[...][None, :], 0.0)
    h1_ref[...] = h1
    h1neg_ref[...] = -h1
    h2r = jnp.dot(h1.astype(jnp.bfloat16), w2_ref[...].astype(jnp.bfloat16),
                  preferred_element_type=jnp.float32)
    h2p_ref[...] = h2r * dinv[:, None]


def _tc4_body(p_ref, h2p_ref, dinv_ref, b2_ref, h2_ref, h2neg_ref):
    agg = p_ref[0, :N, :] + p_ref[1, :N, :] + h2p_ref[...]
    h2 = agg * dinv_ref[...][:, None] + b2_ref[...][None, :]
    h2_ref[...] = h2
    h2neg_ref[...] = -h2


_tc1 = pl.pallas_call(
    _tc1_body,
    out_shape=(jax.ShapeDtypeStruct((N, HID), jnp.float32),
               jax.ShapeDtypeStruct((N, D_IN), jnp.float32),
               jax.ShapeDtypeStruct((N,), jnp.float32)),
)

_tc3 = pl.pallas_call(
    _tc3_body,
    out_shape=(jax.ShapeDtypeStruct((N, HID), jnp.float32),
               jax.ShapeDtypeStruct((N, HID), jnp.float32),
               jax.ShapeDtypeStruct((N, NCLS), jnp.float32)),
)

_tc4 = pl.pallas_call(
    _tc4_body,
    out_shape=(jax.ShapeDtypeStruct((N, NCLS), jnp.float32),
               jax.ShapeDtypeStruct((N, NCLS), jnp.float32)),
)

EB = 4096
NB = TE_PAD // EB


def _tc5_body(d1_ref, d2_ref, d3_ref, s1w_ref, s2w_ref, s3w_ref, s3b_ref,
              o3_ref, sum_ref):
    b = pl.program_id(0)
    d1 = d1_ref[...].astype(jnp.bfloat16)
    d2 = d2_ref[...].astype(jnp.bfloat16)
    d3 = d3_ref[...].astype(jnp.bfloat16)
    w1 = s1w_ref[...].astype(jnp.bfloat16)[:, None]
    w2 = s2w_ref[...].astype(jnp.bfloat16)[:, None]
    w3 = s3w_ref[...].astype(jnp.bfloat16)[:, None]
    e1 = jnp.dot(d1, w1, preferred_element_type=jnp.float32)[:, 0]
    e2 = jnp.dot(d2, w2, preferred_element_type=jnp.float32)[:, 0]
    e3 = jnp.dot(d3, w3, preferred_element_type=jnp.float32)[:, 0]
    o3_ref[...] = jnp.maximum(e3 + s3b_ref[...], 0.0)
    gid = b * EB + lax.broadcasted_iota(jnp.int32, (EB,), 0)
    part = jnp.sum(jnp.where(gid < TE, e1 + e2 + e3, 0.0))

    @pl.when(b == 0)
    def _():
        sum_ref[...] = jnp.zeros_like(sum_ref)

    sum_ref[...] += jnp.full((8, 128), part, jnp.float32)


_tc5 = pl.pallas_call(
    _tc5_body,
    grid=(NB,),
    in_specs=[
        pl.BlockSpec((EB, HID), lambda b: (b, 0)),
        pl.BlockSpec((EB, HID), lambda b: (b, 0)),
        pl.BlockSpec((EB, NCLS), lambda b: (b, 0)),
        pl.BlockSpec((D_IN,), lambda b: (0,)),
        pl.BlockSpec((HID,), lambda b: (0,)),
        pl.BlockSpec((NCLS,), lambda b: (0,)),
        pl.BlockSpec((1,), lambda b: (0,)),
    ],
    out_specs=(pl.BlockSpec((EB,), lambda b: (b,)),
               pl.BlockSpec((8, 128), lambda b: (0, 0))),
    out_shape=(jax.ShapeDtypeStruct((TE_PAD,), jnp.float32),
               jax.ShapeDtypeStruct((8, 128), jnp.float32)),
)


def kernel(x, masked_nodes, pos_edge_index, neg_edge_index, edge_index,
           W1, b1, W2, b2, s1w, s1b, s2w, s2b, s3w, s3b):
    src = edge_index[0].astype(jnp.int32)
    dst = edge_index[1].astype(jnp.int32)
    pad_ar = jnp.arange(E_PAD - E, dtype=jnp.int32)
    pad_src = pad_ar % N
    pad_dst = N + pad_ar % (N_PAD - N)
    src_p = jnp.concatenate([src, pad_src]).reshape(NC, NS, ECH, CHUNK)
    dst_flat = jnp.concatenate([dst, pad_dst])
    dst_p = dst_flat.reshape(NC, NS, ECH, CHUNK)
    pad_t = jnp.arange(TE_PAD - TE, dtype=jnp.int32) % N
    t0 = jnp.concatenate(
        [pos_edge_index[0], neg_edge_index[0], pad_t]).astype(jnp.int32)
    t1 = jnp.concatenate(
        [pos_edge_index[1], neg_edge_index[1], pad_t]).astype(jnp.int32)
    t0_p = t0.reshape(NC, NS, ECH, CHUNK)
    t1_p = t1.reshape(NC, NS, ECH, CHUNK)

    cnt = _deg_kernel(dst_flat).reshape(NW, N_PAD)
    hp, xneg, dinv = _tc1(x, W1, cnt)
    dist1 = _dist_hid(x, xneg, t0_p, t1_p)
    parts1 = _gs_hid(src_p, dst_p, hp, jnp.zeros((N_PAD, HID), jnp.float32))
    h1, h1neg, h2p = _tc3(parts1, hp, dinv, b1, W2)
    dist2 = _dist_hid(h1, h1neg, t0_p, t1_p)
    parts2 = _gs_cls(src_p, dst_p, h2p, jnp.zeros((N_PAD, NCLS), jnp.float32))
    h2, h2neg = _tc4(parts2, h2p, dinv, b2)
    dist3 = _dist_cls(h2, h2neg, t0_p, t1_p)
    o3_pad, sums = _tc5(dist1, dist2, dist3, s1w, s2w, s3w, s3b)
    score_loss = sums[0, 0] / TE
    return (o3_pad[:TE], score_loss)


# speedup vs baseline: 3.1884x; 1.1611x over previous
import functools

import jax
import jax.numpy as jnp
from jax import lax
from jax.experimental import pallas as pl
from jax.experimental.pallas import tpu as pltpu
from jax.experimental.pallas import tpu_sc as plsc

N = 10000
D_IN = 128
HID = 128
NCLS = 64
E = 320000
TE = 320000

NC = 2
NS = 16
NW = NC * NS
CHUNK = 128
ECH = 80
KBUF = 2
E_PAD = NW * ECH * CHUNK
N_PAD = 10112
ROWS_PER_TILE = N_PAD // NS
E_PER_TILE = E_PAD // NW
TE_PAD = E_PAD
TE_PER_TILE = TE_PAD // NW

_MESH = plsc.VectorSubcoreMesh(
    core_axis_name="c", subcore_axis_name="s", num_cores=NC, num_subcores=NS)

_SC_PARAMS = pltpu.CompilerParams(needs_layout_passes=False,
                                  use_tc_tiling_on_sc=False)


@functools.partial(
    pl.kernel,
    out_type=jax.ShapeDtypeStruct((NW * N_PAD,), jnp.float32),
    mesh=_MESH,
    compiler_params=_SC_PARAMS,
    scratch_types=[
        pltpu.VMEM((E_PER_TILE,), jnp.int32),
        pltpu.VMEM((N_PAD,), jnp.float32),
    ],
)
def _deg_kernel(dst_hbm, out_hbm, idx_v, cnt_v):
    c = lax.axis_index("c")
    s = lax.axis_index("s")
    w = c * NS + s
    pltpu.sync_copy(dst_hbm.at[pl.ds(w * E_PER_TILE, E_PER_TILE)], idx_v)
    zero16 = jnp.zeros((16,), jnp.float32)

    def zbody(i, _):
        cnt_v[pl.ds(i * 16, 16)] = zero16
        return 0

    lax.fori_loop(0, N_PAD // 16, zbody, 0)
    one16 = jnp.ones((16,), jnp.float32)

    def body(i, _):
        idx = idx_v[pl.ds(i * 16, 16)]
        plsc.addupdate_scatter(cnt_v, [idx], one16)
        return 0

    lax.fori_loop(0, E_PER_TILE // 16, body, 0)
    pltpu.sync_copy(cnt_v, out_hbm.at[pl.ds(w * N_PAD, N_PAD)])


def _make_gather_scatter(D, nbuf):
    scratch = ([pltpu.VMEM((ECH, CHUNK), jnp.int32),
                pltpu.VMEM((ECH, CHUNK), jnp.int32)]
               + [pltpu.VMEM((CHUNK, D), jnp.float32) for _ in range(nbuf)]
               + [pltpu.VMEM_SHARED((N_PAD, D), jnp.float32)]
               + [pltpu.SemaphoreType.DMA for _ in range(2 * nbuf)])

    @functools.partial(
        pl.kernel,
        out_type=jax.ShapeDtypeStruct((NC, N_PAD, D), jnp.float32),
        mesh=_MESH,
        compiler_params=_SC_PARAMS,
        scratch_types=scratch,
    )
    def k(src_hbm, dst_hbm, h_hbm, zeros_hbm, out_hbm, src_v, dst_v, *rest):
        rows = rest[:nbuf]
        acc = rest[nbuf]
        sems = rest[nbuf + 1:]
        c = lax.axis_index("c")
        s = lax.axis_index("s")
        pltpu.sync_copy(src_hbm.at[c, s], src_v)
        pltpu.sync_copy(dst_hbm.at[c, s], dst_v)
        row0 = s * ROWS_PER_TILE
        pltpu.sync_copy(zeros_hbm.at[pl.ds(row0, ROWS_PER_TILE)],
                        acc.at[pl.ds(row0, ROWS_PER_TILE)])
        plsc.subcore_barrier()

        def body(j, _):
            pltpu.async_copy(h_hbm.at[src_v.at[j]], rows[0], sems[0]).wait()
            pltpu.sync_copy(rows[0], acc.at[dst_v.at[j]], add=True)
            return 0

        lax.fori_loop(0, ECH, body, 0)
        plsc.subcore_barrier()
        pltpu.sync_copy(acc.at[pl.ds(row0, ROWS_PER_TILE)],
                        out_hbm.at[c, pl.ds(row0, ROWS_PER_TILE)])

    return k


_gs_hid = _make_gather_scatter(HID, 1)
_gs_cls = _make_gather_scatter(NCLS, 1)


def _make_dist(D):
    scratch = ([pltpu.VMEM((ECH, CHUNK), jnp.int32),
                pltpu.VMEM((ECH, CHUNK), jnp.int32)]
               + [pltpu.VMEM((CHUNK, D), jnp.float32) for _ in range(KBUF)]
               + [pltpu.SemaphoreType.DMA for _ in range(3 * KBUF)])

    @functools.partial(
        pl.kernel,
        out_type=jax.ShapeDtypeStruct((TE_PAD, D), jnp.float32),
        mesh=_MESH,
        compiler_params=_SC_PARAMS,
        scratch_types=scratch,
    )
    def k(pos_hbm, neg_hbm, t0_hbm, t1_hbm, out_hbm, t0_v, t1_v, *rest):
        rows = rest[:KBUF]
        semg = rest[KBUF:2 * KBUF]
        sema = rest[2 * KBUF:3 * KBUF]
        semw = rest[3 * KBUF:4 * KBUF]
        c = lax.axis_index("c")
        s = lax.axis_index("s")
        w = c * NS + s
        base = w * E_PER_TILE

        pltpu.sync_copy(t0_hbm.at[c, s], t0_v)
        pltpu.sync_copy(t1_hbm.at[c, s], t1_v)

        def body(p, _):
            jb = p * KBUF
            cg = [pltpu.async_copy(pos_hbm.at[t0_v.at[jb + b]], rows[b],
                                   semg[b]) for b in range(KBUF)]
            ca = []
            for b in range(KBUF):
                cg[b].wait()
                ca.append(pltpu.async_copy(neg_hbm.at[t1_v.at[jb + b]],
                                           rows[b], sema[b], add=True))
            cw = []
            for b in range(KBUF):
                ca[b].wait()
                cw.append(pltpu.async_copy(
                    rows[b],
                    out_hbm.at[pl.ds(base + (jb + b) * CHUNK, CHUNK)],
                    semw[b]))
            for cp in cw:
                cp.wait()
            return 0

        lax.fori_loop(0, ECH // KBUF, body, 0)

    return k


_dist_hid = _make_dist(HID)
_dist_cls = _make_dist(NCLS)


def _tc1_body(x_ref, w1_ref, cnt_ref, hp_ref, xneg_ref, dinv_ref):
    cs = jnp.sum(cnt_ref[...], axis=0)
    deg = cs[:N] + 1.0
    y = lax.rsqrt(deg)
    dinv = y * (1.5 - 0.5 * deg * y * y)
    dinv_ref[...] = dinv
    xx = x_ref[...]
    xneg_ref[...] = -xx
    h = jnp.dot(xx.astype(jnp.bfloat16), w1_ref[...].astype(jnp.bfloat16),
                preferred_element_type=jnp.float32)
    hp_ref[...] = h * dinv[:, None]


def _tc3_body(p_ref, hp_ref, dinv_ref, b1_ref, w2_ref,
              h1_ref, h1neg_ref, h2p_ref):
    agg = p_ref[0, :N, :] + p_ref[1, :N, :] + hp_ref[...]
    dinv = dinv_ref[...]
    h1 = jnp.maximum(agg * dinv[:, None] + b1_ref[...][None, :], 0.0)
    h1_ref[...] = h1
    h1neg_ref[...] = -h1
    h2r = jnp.dot(h1.astype(jnp.bfloat16), w2_ref[...].astype(jnp.bfloat16),
                  preferred_element_type=jnp.float32)
    h2p_ref[...] = h2r * dinv[:, None]


def _tc4_body(p_ref, h2p_ref, dinv_ref, b2_ref, h2_ref, h2neg_ref):
    agg = p_ref[0, :N, :] + p_ref[1, :N, :] + h2p_ref[...]
    h2 = agg * dinv_ref[...][:, None] + b2_ref[...][None, :]
    h2_ref[...] = h2
    h2neg_ref[...] = -h2


_tc1 = pl.pallas_call(
    _tc1_body,
    out_shape=(jax.ShapeDtypeStruct((N, HID), jnp.float32),
               jax.ShapeDtypeStruct((N, D_IN), jnp.float32),
               jax.ShapeDtypeStruct((N,), jnp.float32)),
)

_tc3 = pl.pallas_call(
    _tc3_body,
    out_shape=(jax.ShapeDtypeStruct((N, HID), jnp.float32),
               jax.ShapeDtypeStruct((N, HID), jnp.float32),
               jax.ShapeDtypeStruct((N, NCLS), jnp.float32)),
)

_tc4 = pl.pallas_call(
    _tc4_body,
    out_shape=(jax.ShapeDtypeStruct((N, NCLS), jnp.float32),
               jax.ShapeDtypeStruct((N, NCLS), jnp.float32)),
)

EB = 4096
NB = TE_PAD // EB


def _tc5_body(d1_ref, d2_ref, d3_ref, s1w_ref, s2w_ref, s3w_ref, s3b_ref,
              o3_ref, sum_ref):
    b = pl.program_id(0)
    d1 = d1_ref[...].astype(jnp.bfloat16)
    d2 = d2_ref[...].astype(jnp.bfloat16)
    d3 = d3_ref[...].astype(jnp.bfloat16)
    w1 = s1w_ref[...].astype(jnp.bfloat16)[:, None]
    w2 = s2w_ref[...].astype(jnp.bfloat16)[:, None]
    w3 = s3w_ref[...].astype(jnp.bfloat16)[:, None]
    e1 = jnp.dot(d1, w1, preferred_element_type=jnp.float32)[:, 0]
    e2 = jnp.dot(d2, w2, preferred_element_type=jnp.float32)[:, 0]
    e3 = jnp.dot(d3, w3, preferred_element_type=jnp.float32)[:, 0]
    o3_ref[...] = jnp.maximum(e3 + s3b_ref[...], 0.0)
    gid = b * EB + lax.broadcasted_iota(jnp.int32, (EB,), 0)
    part = jnp.sum(jnp.where(gid < TE, e1 + e2 + e3, 0.0))

    @pl.when(b == 0)
    def _():
        sum_ref[...] = jnp.zeros_like(sum_ref)

    sum_ref[...] += jnp.full((8, 128), part, jnp.float32)


_tc5 = pl.pallas_call(
    _tc5_body,
    grid=(NB,),
    in_specs=[
        pl.BlockSpec((EB, HID), lambda b: (b, 0)),
        pl.BlockSpec((EB, HID), lambda b: (b, 0)),
        pl.BlockSpec((EB, NCLS), lambda b: (b, 0)),
        pl.BlockSpec((D_IN,), lambda b: (0,)),
        pl.BlockSpec((HID,), lambda b: (0,)),
        pl.BlockSpec((NCLS,), lambda b: (0,)),
        pl.BlockSpec((1,), lambda b: (0,)),
    ],
    out_specs=(pl.BlockSpec((EB,), lambda b: (b,)),
               pl.BlockSpec((8, 128), lambda b: (0, 0))),
    out_shape=(jax.ShapeDtypeStruct((TE_PAD,), jnp.float32),
               jax.ShapeDtypeStruct((8, 128), jnp.float32)),
)


def kernel(x, masked_nodes, pos_edge_index, neg_edge_index, edge_index,
           W1, b1, W2, b2, s1w, s1b, s2w, s2b, s3w, s3b):
    src = edge_index[0].astype(jnp.int32)
    dst = edge_index[1].astype(jnp.int32)
    pad_ar = jnp.arange(E_PAD - E, dtype=jnp.int32)
    pad_src = pad_ar % N
    pad_dst = N + pad_ar % (N_PAD - N)
    src_p = jnp.concatenate([src, pad_src]).reshape(NC, NS, ECH, CHUNK)
    dst_flat = jnp.concatenate([dst, pad_dst])
    dst_p = dst_flat.reshape(NC, NS, ECH, CHUNK)
    pad_t = jnp.arange(TE_PAD - TE, dtype=jnp.int32) % N
    t0 = jnp.concatenate(
        [pos_edge_index[0], neg_edge_index[0], pad_t]).astype(jnp.int32)
    t1 = jnp.concatenate(
        [pos_edge_index[1], neg_edge_index[1], pad_t]).astype(jnp.int32)
    t0_p = t0.reshape(NC, NS, ECH, CHUNK)
    t1_p = t1.reshape(NC, NS, ECH, CHUNK)

    cnt = _deg_kernel(dst_flat).reshape(NW, N_PAD)
    hp, xneg, dinv = _tc1(x, W1, cnt)
    dist1 = _dist_hid(x, xneg, t0_p, t1_p)
    parts1 = _gs_hid(src_p, dst_p, hp, jnp.zeros((N_PAD, HID), jnp.float32))
    h1, h1neg, h2p = _tc3(parts1, hp, dinv, b1, W2)
    dist2 = _dist_hid(h1, h1neg, t0_p, t1_p)
    parts2 = _gs_cls(src_p, dst_p, h2p, jnp.zeros((N_PAD, NCLS), jnp.float32))
    h2, h2neg = _tc4(parts2, h2p, dinv, b2)
    dist3 = _dist_cls(h2, h2neg, t0_p, t1_p)
    o3_pad, sums = _tc5(dist1, dist2, dist3, s1w, s2w, s3w, s3b)
    score_loss = sums[0, 0] / TE
    return (o3_pad[:TE], score_loss)


# speedup vs baseline: 3.4137x; 1.0707x over previous
import functools

import jax
import jax.numpy as jnp
from jax import lax
from jax.experimental import pallas as pl
from jax.experimental.pallas import tpu as pltpu
from jax.experimental.pallas import tpu_sc as plsc

N = 10000
D_IN = 128
HID = 128
NCLS = 64
E = 320000
TE = 320000

NC = 2
NS = 16
NW = NC * NS
CHUNK = 128
ECH = 80
KBUF = 4
E_PAD = NW * ECH * CHUNK
N_PAD = 10112
ROWS_PER_TILE = N_PAD // NS
E_PER_TILE = E_PAD // NW
TE_PAD = E_PAD
TE_PER_TILE = TE_PAD // NW

_MESH = plsc.VectorSubcoreMesh(
    core_axis_name="c", subcore_axis_name="s", num_cores=NC, num_subcores=NS)

_SC_PARAMS = pltpu.CompilerParams(needs_layout_passes=False,
                                  use_tc_tiling_on_sc=False)


@functools.partial(
    pl.kernel,
    out_type=jax.ShapeDtypeStruct((NW * N_PAD,), jnp.float32),
    mesh=_MESH,
    compiler_params=_SC_PARAMS,
    scratch_types=[
        pltpu.VMEM((E_PER_TILE,), jnp.int32),
        pltpu.VMEM((N_PAD,), jnp.float32),
    ],
)
def _deg_kernel(dst_hbm, out_hbm, idx_v, cnt_v):
    c = lax.axis_index("c")
    s = lax.axis_index("s")
    w = c * NS + s
    pltpu.sync_copy(dst_hbm.at[pl.ds(w * E_PER_TILE, E_PER_TILE)], idx_v)
    zero16 = jnp.zeros((16,), jnp.float32)

    def zbody(i, _):
        cnt_v[pl.ds(i * 16, 16)] = zero16
        return 0

    lax.fori_loop(0, N_PAD // 16, zbody, 0)
    one16 = jnp.ones((16,), jnp.float32)

    def body(i, _):
        idx = idx_v[pl.ds(i * 16, 16)]
        plsc.addupdate_scatter(cnt_v, [idx], one16)
        return 0

    lax.fori_loop(0, E_PER_TILE // 16, body, 0)
    pltpu.sync_copy(cnt_v, out_hbm.at[pl.ds(w * N_PAD, N_PAD)])


def _make_gather_scatter(D, nbuf):
    scratch = ([pltpu.VMEM((ECH, CHUNK), jnp.int32),
                pltpu.VMEM((ECH, CHUNK), jnp.int32)]
               + [pltpu.VMEM((CHUNK, D), jnp.float32) for _ in range(nbuf)]
               + [pltpu.VMEM_SHARED((N_PAD, D), jnp.float32)]
               + [pltpu.SemaphoreType.DMA for _ in range(2 * nbuf)])

    @functools.partial(
        pl.kernel,
        out_type=jax.ShapeDtypeStruct((NC, N_PAD, D), jnp.float32),
        mesh=_MESH,
        compiler_params=_SC_PARAMS,
        scratch_types=scratch,
    )
    def k(src_hbm, dst_hbm, h_hbm, zeros_hbm, out_hbm, src_v, dst_v, *rest):
        rows = rest[:nbuf]
        acc = rest[nbuf]
        sems = rest[nbuf + 1:]
        c = lax.axis_index("c")
        s = lax.axis_index("s")
        pltpu.sync_copy(src_hbm.at[c, s], src_v)
        pltpu.sync_copy(dst_hbm.at[c, s], dst_v)
        row0 = s * ROWS_PER_TILE
        pltpu.sync_copy(zeros_hbm.at[pl.ds(row0, ROWS_PER_TILE)],
                        acc.at[pl.ds(row0, ROWS_PER_TILE)])
        plsc.subcore_barrier()

        def body(p, _):
            jb = p * nbuf
            gs = [pltpu.async_copy(h_hbm.at[src_v.at[jb + b]], rows[b],
                                   sems[b]) for b in range(nbuf)]
            ss = []
            for b in range(nbuf):
                gs[b].wait()
                ss.append(pltpu.async_copy(rows[b], acc.at[dst_v.at[jb + b]],
                                           sems[nbuf + b], add=True))
            for cp in ss:
                cp.wait()
            return 0

        lax.fori_loop(0, ECH // nbuf, body, 0)
        plsc.subcore_barrier()
        pltpu.sync_copy(acc.at[pl.ds(row0, ROWS_PER_TILE)],
                        out_hbm.at[c, pl.ds(row0, ROWS_PER_TILE)])

    return k


_gs_hid = _make_gather_scatter(HID, 1)
_gs_cls = _make_gather_scatter(NCLS, 2)


def _make_dist(D):
    scratch = ([pltpu.VMEM((ECH, CHUNK), jnp.int32),
                pltpu.VMEM((ECH, CHUNK), jnp.int32)]
               + [pltpu.VMEM((CHUNK, D), jnp.float32) for _ in range(KBUF)]
               + [pltpu.SemaphoreType.DMA for _ in range(3 * KBUF)])

    @functools.partial(
        pl.kernel,
        out_type=jax.ShapeDtypeStruct((TE_PAD, D), jnp.float32),
        mesh=_MESH,
        compiler_params=_SC_PARAMS,
        scratch_types=scratch,
    )
    def k(pos_hbm, neg_hbm, t0_hbm, t1_hbm, out_hbm, t0_v, t1_v, *rest):
        rows = rest[:KBUF]
        semg = rest[KBUF:2 * KBUF]
        sema = rest[2 * KBUF:3 * KBUF]
        semw = rest[3 * KBUF:4 * KBUF]
        c = lax.axis_index("c")
        s = lax.axis_index("s")
        w = c * NS + s
        base = w * E_PER_TILE

        pltpu.sync_copy(t0_hbm.at[c, s], t0_v)
        pltpu.sync_copy(t1_hbm.at[c, s], t1_v)

        def body(p, _):
            jb = p * KBUF
            cg = [pltpu.async_copy(pos_hbm.at[t0_v.at[jb + b]], rows[b],
                                   semg[b]) for b in range(KBUF)]
            ca = []
            for b in range(KBUF):
                cg[b].wait()
                ca.append(pltpu.async_copy(neg_hbm.at[t1_v.at[jb + b]],
                                           rows[b], sema[b], add=True))
            cw = []
            for b in range(KBUF):
                ca[b].wait()
                cw.append(pltpu.async_copy(
                    rows[b],
                    out_hbm.at[pl.ds(base + (jb + b) * CHUNK, CHUNK)],
                    semw[b]))
            for cp in cw:
                cp.wait()
            return 0

        lax.fori_loop(0, ECH // KBUF, body, 0)

    return k


_dist_hid = _make_dist(HID)
_dist_cls = _make_dist(NCLS)


def _tc1_body(x_ref, w1_ref, cnt_ref, hp_ref, xneg_ref, dinv_ref):
    cs = jnp.sum(cnt_ref[...], axis=0)
    deg = cs[:N] + 1.0
    y = lax.rsqrt(deg)
    dinv = y * (1.5 - 0.5 * deg * y * y)
    dinv_ref[...] = dinv
    xx = x_ref[...]
    xneg_ref[...] = -xx
    h = jnp.dot(xx.astype(jnp.bfloat16), w1_ref[...].astype(jnp.bfloat16),
                preferred_element_type=jnp.float32)
    hp_ref[...] = h * dinv[:, None]


def _tc3_body(p_ref, hp_ref, dinv_ref, b1_ref, w2_ref,
              h1_ref, h1neg_ref, h2p_ref):
    agg = p_ref[0, :N, :] + p_ref[1, :N, :] + hp_ref[...]
    dinv = dinv_ref[...]
    h1 = jnp.maximum(agg * dinv[:, None] + b1_ref[...][None, :], 0.0)
    h1_ref[...] = h1
    h1neg_ref[...] = -h1
    h2r = jnp.dot(h1.astype(jnp.bfloat16), w2_ref[...].astype(jnp.bfloat16),
                  preferred_element_type=jnp.float32)
    h2p_ref[...] = h2r * dinv[:, None]


def _tc4_body(p_ref, h2p_ref, dinv_ref, b2_ref, h2_ref, h2neg_ref):
    agg = p_ref[0, :N, :] + p_ref[1, :N, :] + h2p_ref[...]
    h2 = agg * dinv_ref[...][:, None] + b2_ref[...][None, :]
    h2_ref[...] = h2
    h2neg_ref[...] = -h2


_tc1 = pl.pallas_call(
    _tc1_body,
    out_shape=(jax.ShapeDtypeStruct((N, HID), jnp.float32),
               jax.ShapeDtypeStruct((N, D_IN), jnp.float32),
               jax.ShapeDtypeStruct((N,), jnp.float32)),
)

_tc3 = pl.pallas_call(
    _tc3_body,
    out_shape=(jax.ShapeDtypeStruct((N, HID), jnp.float32),
               jax.ShapeDtypeStruct((N, HID), jnp.float32),
               jax.ShapeDtypeStruct((N, NCLS), jnp.float32)),
)

_tc4 = pl.pallas_call(
    _tc4_body,
    out_shape=(jax.ShapeDtypeStruct((N, NCLS), jnp.float32),
               jax.ShapeDtypeStruct((N, NCLS), jnp.float32)),
)

EB = 4096
NB = TE_PAD // EB


def _tc5_body(d1_ref, d2_ref, d3_ref, s1w_ref, s2w_ref, s3w_ref, s3b_ref,
              o3_ref, sum_ref):
    b = pl.program_id(0)
    d1 = d1_ref[...].astype(jnp.bfloat16)
    d2 = d2_ref[...].astype(jnp.bfloat16)
    d3 = d3_ref[...].astype(jnp.bfloat16)
    w1 = s1w_ref[...].astype(jnp.bfloat16)[:, None]
    w2 = s2w_ref[...].astype(jnp.bfloat16)[:, None]
    w3 = s3w_ref[...].astype(jnp.bfloat16)[:, None]
    e1 = jnp.dot(d1, w1, preferred_element_type=jnp.float32)[:, 0]
    e2 = jnp.dot(d2, w2, preferred_element_type=jnp.float32)[:, 0]
    e3 = jnp.dot(d3, w3, preferred_element_type=jnp.float32)[:, 0]
    o3_ref[...] = jnp.maximum(e3 + s3b_ref[...], 0.0)
    gid = b * EB + lax.broadcasted_iota(jnp.int32, (EB,), 0)
    part = jnp.sum(jnp.where(gid < TE, e1 + e2 + e3, 0.0))

    @pl.when(b == 0)
    def _():
        sum_ref[...] = jnp.zeros_like(sum_ref)

    sum_ref[...] += jnp.full((8, 128), part, jnp.float32)


_tc5 = pl.pallas_call(
    _tc5_body,
    grid=(NB,),
    in_specs=[
        pl.BlockSpec((EB, HID), lambda b: (b, 0)),
        pl.BlockSpec((EB, HID), lambda b: (b, 0)),
        pl.BlockSpec((EB, NCLS), lambda b: (b, 0)),
        pl.BlockSpec((D_IN,), lambda b: (0,)),
        pl.BlockSpec((HID,), lambda b: (0,)),
        pl.BlockSpec((NCLS,), lambda b: (0,)),
        pl.BlockSpec((1,), lambda b: (0,)),
    ],
    out_specs=(pl.BlockSpec((EB,), lambda b: (b,)),
               pl.BlockSpec((8, 128), lambda b: (0, 0))),
    out_shape=(jax.ShapeDtypeStruct((TE_PAD,), jnp.float32),
               jax.ShapeDtypeStruct((8, 128), jnp.float32)),
)


def kernel(x, masked_nodes, pos_edge_index, neg_edge_index, edge_index,
           W1, b1, W2, b2, s1w, s1b, s2w, s2b, s3w, s3b):
    src = edge_index[0].astype(jnp.int32)
    dst = edge_index[1].astype(jnp.int32)
    pad_ar = jnp.arange(E_PAD - E, dtype=jnp.int32)
    pad_src = pad_ar % N
    pad_dst = N + pad_ar % (N_PAD - N)
    src_p = jnp.concatenate([src, pad_src]).reshape(NC, NS, ECH, CHUNK)
    dst_flat = jnp.concatenate([dst, pad_dst])
    dst_p = dst_flat.reshape(NC, NS, ECH, CHUNK)
    pad_t = jnp.arange(TE_PAD - TE, dtype=jnp.int32) % N
    t0 = jnp.concatenate(
        [pos_edge_index[0], neg_edge_index[0], pad_t]).astype(jnp.int32)
    t1 = jnp.concatenate(
        [pos_edge_index[1], neg_edge_index[1], pad_t]).astype(jnp.int32)
    t0_p = t0.reshape(NC, NS, ECH, CHUNK)
    t1_p = t1.reshape(NC, NS, ECH, CHUNK)

    cnt = _deg_kernel(dst_flat).reshape(NW, N_PAD)
    hp, xneg, dinv = _tc1(x, W1, cnt)
    dist1 = _dist_hid(x, xneg, t0_p, t1_p)
    parts1 = _gs_hid(src_p, dst_p, hp, jnp.zeros((N_PAD, HID), jnp.float32))
    h1, h1neg, h2p = _tc3(parts1, hp, dinv, b1, W2)
    dist2 = _dist_hid(h1, h1neg, t0_p, t1_p)
    parts2 = _gs_cls(src_p, dst_p, h2p, jnp.zeros((N_PAD, NCLS), jnp.float32))
    h2, h2neg = _tc4(parts2, h2p, dinv, b2)
    dist3 = _dist_cls(h2, h2neg, t0_p, t1_p)
    o3_pad, sums = _tc5(dist1, dist2, dist3, s1w, s2w, s3w, s3b)
    score_loss = sums[0, 0] / TE
    return (o3_pad[:TE], score_loss)
